# baseline (speedup 1.0000x reference)
import jax
import jax.numpy as jnp
from jax.experimental import pallas as pl
from jax.experimental.pallas import tpu as pltpu

_LANES = 128
_ROWS_PER_BLOCK = 2048


def _roll2(a):
    return jnp.roll(a, -2, axis=1)


def _roll1(a):
    return jnp.roll(a, -1, axis=1)


def _fcos_body(x_ref, tg1_ref, rp_ref, rt_ref, tg4_ref, cn4_ref, out_ref):
    f32 = jnp.float32

    x = x_ref[...]
    tg1 = tg1_ref[...]
    cls_iota = jax.lax.broadcasted_iota(jnp.int32, x.shape, 1)
    m = cls_iota == (tg1 - 1)
    e = jnp.exp(-jnp.abs(x))
    u = 1.0 / (1.0 + e)
    v = e * u
    a = u * u
    b = v * v
    s = x >= 0
    w1 = jnp.where(s, a, b)
    w2 = (a + b) - w1
    lg = jnp.log1p(e)
    mx = jnp.maximum(x, 0.0)
    ce0 = mx + lg
    ce1 = (mx - x) + lg
    fsum = jnp.sum(jnp.where(m, 0.25 * ce1 * w2, 0.75 * ce0 * w1))

    rp = rp_ref[...]
    rt = rt_ref[...]
    tg4 = tg4_ref[...]
    cn4 = cn4_ref[...]

    lane = jax.lax.broadcasted_iota(jnp.int32, rp.shape, 1)
    c = jax.lax.rem(lane, 4)
    m0 = (c == 0) & (tg4 > 0)
    m01 = c < 2

    npos = jnp.sum(jnp.where(m0, 1.0, 0.0))

    rt2 = _roll2(rt)
    ratio = jnp.minimum(rt, rt2) / jnp.maximum(rt, rt2)
    cness_t = jnp.sqrt(ratio * _roll1(ratio))

    xyp = jnp.where(m01, -rp, rp)
    xyt = jnp.where(m01, -rt, rt)
    mn = jnp.minimum(xyp, xyt)
    mx2 = jnp.maximum(xyp, xyt)
    ic = jnp.where(m01, mx2, mn)
    wh = _roll2(ic) - ic
    wh2 = _roll1(wh)
    inter = jnp.where((wh > 0) & (wh2 > 0), wh * wh2, 0.0)
    dp = _roll2(xyp) - xyp
    dt = _roll2(xyt) - xyt
    area_p = dp * _roll1(dp)
    area_t = dt * _roll1(dt)
    union = area_p + area_t - inter
    iou = inter / (union + 1e-7)
    ec = jnp.where(m01, mn, mx2)
    ed = _roll2(ec) - ec
    ed2 = _roll1(ed)
    diag = ed * ed + ed2 * ed2 + 1e-7
    cd = (xyp + _roll2(xyp)) - (xyt + _roll2(xyt))
    cd2 = _roll1(cd)
    cdist = 0.25 * (cd * cd + cd2 * cd2)
    diou = 1.0 - iou + cdist / diag
    w = cness_t
    rnum = jnp.sum(jnp.where(m0, diou * w, 0.0))
    rden = jnp.sum(jnp.where(m0, w, 0.0))

    ec4 = jnp.exp(-jnp.abs(cn4))
    bce = jnp.maximum(cn4, 0.0) - cn4 * cness_t + jnp.log1p(ec4)
    csum = jnp.sum(jnp.where(m0, bce, 0.0))

    out_ref[0, 0, 0] = fsum.astype(f32)
    out_ref[0, 0, 1] = npos
    out_ref[0, 0, 2] = rnum
    out_ref[0, 0, 3] = rden
    out_ref[0, 0, 4] = csum
    out_ref[0, 0, 5] = 0.0
    out_ref[0, 0, 6] = 0.0
    out_ref[0, 0, 7] = 0.0


def kernel(cls_logits, reg_preds, cness_preds, cls_tgts, reg_tgts):
    B, N, C = cls_logits.shape
    BN = B * N
    R = _ROWS_PER_BLOCK
    assert BN % R == 0 and (4 * R) % _LANES == 0
    grid = BN // R
    S4 = (4 * R) // _LANES

    x = cls_logits.reshape(BN, C)
    tg1 = cls_tgts.reshape(BN, 1).astype(jnp.int32)
    rp4 = reg_preds.reshape((4 * BN) // _LANES, _LANES)
    rt4 = reg_tgts.reshape((4 * BN) // _LANES, _LANES)
    tg4 = jnp.broadcast_to(tg1, (BN, 4)).reshape((4 * BN) // _LANES, _LANES)
    cn4 = jnp.broadcast_to(cness_preds.reshape(BN, 1), (BN, 4)) \
             .reshape((4 * BN) // _LANES, _LANES)

    partials = pl.pallas_call(
        _fcos_body,
        grid=(grid,),
        in_specs=[
            pl.BlockSpec((R, C), lambda i: (i, 0)),
            pl.BlockSpec((R, 1), lambda i: (i, 0)),
            pl.BlockSpec((S4, _LANES), lambda i: (i, 0)),
            pl.BlockSpec((S4, _LANES), lambda i: (i, 0)),
            pl.BlockSpec((S4, _LANES), lambda i: (i, 0)),
            pl.BlockSpec((S4, _LANES), lambda i: (i, 0)),
        ],
        out_specs=pl.BlockSpec((1, 1, 8), lambda i: (i, 0, 0), memory_space=pltpu.SMEM),
        out_shape=jax.ShapeDtypeStruct((grid, 1, 8), jnp.float32),
        compiler_params=pltpu.CompilerParams(
            dimension_semantics=("parallel",),
        ),
        interpret=False,
    )(x, tg1, rp4, rt4, tg4, cn4)

    partials = partials.sum(axis=(0, 1))
    num_pos = partials[1]
    denom = jnp.maximum(num_pos, 1.0)
    cls_loss = partials[0] / denom
    reg_loss = partials[2] / (partials[3] + 1e-8)
    cness_loss = partials[4] / denom
    return cls_loss, reg_loss, cness_loss, cls_loss + reg_loss + cness_loss


# speedup vs baseline: 1.8768x; 1.8768x over previous
import jax
import jax.numpy as jnp
from jax.experimental import pallas as pl
from jax.experimental.pallas import tpu as pltpu

_LANES = 128
_ROWS_PER_BLOCK = 2048
_SMALL_GRID = 8


def _focal_body(x_ref, tg1_ref, out_ref):
    x = x_ref[...]
    tg1 = tg1_ref[...]
    cls_iota = jax.lax.broadcasted_iota(jnp.int32, x.shape, 1)
    m = cls_iota == (tg1 - 1)
    e = jnp.exp(-jnp.abs(x))
    u = 1.0 / (1.0 + e)
    v = e * u
    a = u * u
    b = v * v
    s = x >= 0
    w1 = jnp.where(s, a, b)
    w2 = (a + b) - w1
    lg = jnp.log1p(e)
    mx = jnp.maximum(x, 0.0)
    ce0 = mx + lg
    ce1 = (mx - x) + lg
    fsum = 0.25 * jnp.sum(jnp.where(m, ce1 * w2, 3.0 * (ce0 * w1)))
    out_ref[0, 0, 0] = fsum


def _boxes_body(tg2_ref, rpt_ref, rtt_ref, cn_ref, out_ref):
    tg2 = tg2_ref[...]
    posf = (tg2 > 0).astype(jnp.float32)
    npos = jnp.sum(posf)

    p0 = rpt_ref[0]; p1 = rpt_ref[1]; p2 = rpt_ref[2]; p3 = rpt_ref[3]
    t0 = rtt_ref[0]; t1 = rtt_ref[1]; t2 = rtt_ref[2]; t3 = rtt_ref[3]
    lr_min = jnp.minimum(t0, t2); lr_max = jnp.maximum(t0, t2)
    tb_min = jnp.minimum(t1, t3); tb_max = jnp.maximum(t1, t3)
    cness_t = jnp.sqrt(lr_min / lr_max * (tb_min / tb_max))

    x1 = -p0; y1 = -p1; x2 = p2; y2 = p3
    x1g = -t0; y1g = -t1; x2g = t2; y2g = t3
    xi1 = jnp.maximum(x1, x1g); yi1 = jnp.maximum(y1, y1g)
    xi2 = jnp.minimum(x2, x2g); yi2 = jnp.minimum(y2, y2g)
    inter = jnp.where((yi2 > yi1) & (xi2 > xi1), (xi2 - xi1) * (yi2 - yi1), 0.0)
    union = (x2 - x1) * (y2 - y1) + (x2g - x1g) * (y2g - y1g) - inter
    iou = inter / (union + 1e-7)
    xc1 = jnp.minimum(x1, x1g); yc1 = jnp.minimum(y1, y1g)
    xc2 = jnp.maximum(x2, x2g); yc2 = jnp.maximum(y2, y2g)
    diag = (xc2 - xc1) ** 2 + (yc2 - yc1) ** 2 + 1e-7
    cdist = ((x1 + x2) / 2.0 - (x1g + x2g) / 2.0) ** 2 + \
            ((y1 + y2) / 2.0 - (y1g + y2g) / 2.0) ** 2
    diou = 1.0 - iou + cdist / diag
    w = cness_t * posf
    rnum = jnp.sum(diou * w)
    rden = jnp.sum(w)

    cn = cn_ref[...]
    bce = jnp.maximum(cn, 0.0) - cn * cness_t + jnp.log1p(jnp.exp(-jnp.abs(cn)))
    csum = jnp.sum(bce * posf)

    out_ref[0, 0, 0] = npos
    out_ref[0, 0, 1] = rnum
    out_ref[0, 0, 2] = rden
    out_ref[0, 0, 3] = csum
    out_ref[0, 0, 4] = 0.0
    out_ref[0, 0, 5] = 0.0
    out_ref[0, 0, 6] = 0.0
    out_ref[0, 0, 7] = 0.0


def kernel(cls_logits, reg_preds, cness_preds, cls_tgts, reg_tgts):
    B, N, C = cls_logits.shape
    BN = B * N
    R = _ROWS_PER_BLOCK
    assert BN % R == 0 and BN % (_SMALL_GRID * _LANES) == 0
    grid = BN // R
    SR = BN // _LANES
    S = SR // _SMALL_GRID

    x = cls_logits.reshape(BN, C)
    tg1 = cls_tgts.reshape(BN, 1).astype(jnp.int32)
    tg2 = cls_tgts.reshape(SR, _LANES).astype(jnp.int32)
    rpt = reg_preds.reshape(BN, 4).T.reshape(4, SR, _LANES)
    rtt = reg_tgts.reshape(BN, 4).T.reshape(4, SR, _LANES)
    cn = cness_preds.reshape(SR, _LANES)

    fpart = pl.pallas_call(
        _focal_body,
        grid=(grid,),
        in_specs=[
            pl.BlockSpec((R, C), lambda i: (i, 0)),
            pl.BlockSpec((R, 1), lambda i: (i, 0)),
        ],
        out_specs=pl.BlockSpec((1, 1, 8), lambda i: (i, 0, 0), memory_space=pltpu.SMEM),
        out_shape=jax.ShapeDtypeStruct((grid, 1, 8), jnp.float32),
        compiler_params=pltpu.CompilerParams(
            dimension_semantics=("parallel",),
        ),
        interpret=False,
    )(x, tg1)

    bpart = pl.pallas_call(
        _boxes_body,
        grid=(_SMALL_GRID,),
        in_specs=[
            pl.BlockSpec((S, _LANES), lambda i: (i, 0)),
            pl.BlockSpec((4, S, _LANES), lambda i: (0, i, 0)),
            pl.BlockSpec((4, S, _LANES), lambda i: (0, i, 0)),
            pl.BlockSpec((S, _LANES), lambda i: (i, 0)),
        ],
        out_specs=pl.BlockSpec((1, 1, 8), lambda i: (i, 0, 0), memory_space=pltpu.SMEM),
        out_shape=jax.ShapeDtypeStruct((_SMALL_GRID, 1, 8), jnp.float32),
        compiler_params=pltpu.CompilerParams(
            dimension_semantics=("parallel",),
        ),
        interpret=False,
    )(tg2, rpt, rtt, cn)

    fsum = fpart[:, 0, 0].sum()
    bsum = bpart.sum(axis=(0, 1))
    num_pos = bsum[0]
    denom = jnp.maximum(num_pos, 1.0)
    cls_loss = fsum / denom
    reg_loss = bsum[1] / (bsum[2] + 1e-8)
    cness_loss = bsum[3] / denom
    return cls_loss, reg_loss, cness_loss, cls_loss + reg_loss + cness_loss
